# baseline (speedup 1.0000x reference)
import functools

import jax
import jax.numpy as jnp
from jax.experimental import pallas as pl
from jax.experimental.pallas import tpu as pltpu

_BM = 400
_VMEM_LIMIT = 64 * 1024 * 1024


def _pass_a_body(x_ref, adj_ref, w1_ref, b1_ref, w2_ref, s2_ref, csum_ref,
                 adjbf_ref, s1_scr):
    i = pl.program_id(0)

    @pl.when(i == 0)
    def _init():
        s1_scr[...] = jnp.dot(x_ref[...], w1_ref[...],
                              preferred_element_type=jnp.float32)

    adj_blk = adj_ref[...]
    h1 = jnp.maximum(
        jnp.dot(adj_blk, s1_scr[...], preferred_element_type=jnp.float32)
        + b1_ref[...], 0.0)
    s2_ref[...] = jnp.dot(h1, w2_ref[...],
                          preferred_element_type=jnp.float32
                          ).astype(jnp.bfloat16)

    adj_bf = adj_blk.astype(jnp.bfloat16)
    adjbf_ref[...] = adj_bf
    ones = jnp.ones((1, adj_bf.shape[0]), dtype=jnp.bfloat16)
    part = jnp.dot(ones, adj_bf, preferred_element_type=jnp.float32)

    @pl.when(i == 0)
    def _set():
        csum_ref[...] = part

    @pl.when(i > 0)
    def _acc():
        csum_ref[...] = csum_ref[...] + part


def _pass_b_body(adj_ref, s2_ref, csum_ref, b2_ref, w3_ref, b3_ref, wl_ref,
                 bl_ref, out_ref, acc_scr, *, n_nodes):
    i = pl.program_id(0)
    h2 = jnp.maximum(
        jnp.dot(adj_ref[...], s2_ref[...],
                preferred_element_type=jnp.float32)
        + b2_ref[...], 0.0)
    contrib = jnp.dot(csum_ref[0], h2, preferred_element_type=jnp.float32)

    @pl.when(i == 0)
    def _set():
        acc_scr[...] = contrib

    @pl.when(i > 0)
    def _acc():
        acc_scr[...] = acc_scr[...] + contrib

    @pl.when(i == pl.num_programs(0) - 1)
    def _emit():
        m = acc_scr[...] * (1.0 / n_nodes)
        s3 = jnp.dot(m, w3_ref[...],
                     preferred_element_type=jnp.float32) + b3_ref[...]
        out_ref[...] = jnp.dot(s3, wl_ref[...],
                               preferred_element_type=jnp.float32) + bl_ref[...]


@jax.jit
def kernel(x, adj, W1, b1, W2, b2, W3, b3, Wl, bl):
    n, nfeat = x.shape
    nhid = W1.shape[1]
    nclasses = Wl.shape[1]
    bm = _BM
    grid = (n // bm,)

    b1r = b1.reshape(1, nhid)
    b2r = b2.reshape(1, nhid)
    b3r = b3.reshape(1, nhid)
    blr = bl.reshape(1, nclasses)

    full = lambda shape: pl.BlockSpec(shape, lambda i: (0,) * len(shape))

    s2, csum, adj_bf = pl.pallas_call(
        _pass_a_body,
        grid=grid,
        in_specs=[
            full((n, nfeat)),
            pl.BlockSpec((bm, n), lambda i: (i, 0)),
            full((nfeat, nhid)),
            full((1, nhid)),
            full((nhid, nhid)),
        ],
        out_specs=[
            pl.BlockSpec((bm, nhid), lambda i: (i, 0)),
            full((1, n)),
            pl.BlockSpec((bm, n), lambda i: (i, 0)),
        ],
        out_shape=[
            jax.ShapeDtypeStruct((n, nhid), jnp.bfloat16),
            jax.ShapeDtypeStruct((1, n), jnp.float32),
            jax.ShapeDtypeStruct((n, n), jnp.bfloat16),
        ],
        scratch_shapes=[pltpu.VMEM((n, nhid), jnp.float32)],
        compiler_params=pltpu.CompilerParams(vmem_limit_bytes=_VMEM_LIMIT),
    )(x, adj, W1, b1r, W2)

    csum3 = csum.reshape(n // bm, 1, bm)

    out = pl.pallas_call(
        functools.partial(_pass_b_body, n_nodes=n),
        grid=grid,
        in_specs=[
            pl.BlockSpec((bm, n), lambda i: (i, 0)),
            full((n, nhid)),
            pl.BlockSpec((1, 1, bm), lambda i: (i, 0, 0)),
            full((1, nhid)),
            full((nhid, nhid)),
            full((1, nhid)),
            full((nhid, nclasses)),
            full((1, nclasses)),
        ],
        out_specs=full((1, nclasses)),
        out_shape=jax.ShapeDtypeStruct((1, nclasses), jnp.float32),
        scratch_shapes=[pltpu.VMEM((1, nhid), jnp.float32)],
        compiler_params=pltpu.CompilerParams(vmem_limit_bytes=_VMEM_LIMIT),
    )(adj_bf, s2, csum3, b2r, W3, b3r, Wl, blr)

    return out


# speedup vs baseline: 1.0159x; 1.0159x over previous
import functools

import jax
import jax.numpy as jnp
from jax.experimental import pallas as pl
from jax.experimental.pallas import tpu as pltpu

_BM = 400
_VMEM_LIMIT = 64 * 1024 * 1024


def _pass_a_body(x_ref, adj_ref, w1_ref, b1_ref, w2_ref, s2_ref, csum_ref,
                 s1_scr):
    i = pl.program_id(0)

    @pl.when(i == 0)
    def _init():
        s1_scr[...] = jnp.dot(x_ref[...], w1_ref[...],
                              preferred_element_type=jnp.float32)

    adj_blk = adj_ref[...]
    h1 = jnp.maximum(
        jnp.dot(adj_blk, s1_scr[...], preferred_element_type=jnp.float32)
        + b1_ref[...], 0.0)
    s2_ref[...] = jnp.dot(h1, w2_ref[...],
                          preferred_element_type=jnp.float32)

    adj_bf = adj_blk.astype(jnp.bfloat16)
    ones = jnp.ones((1, adj_bf.shape[0]), dtype=jnp.bfloat16)
    part = jnp.dot(ones, adj_bf, preferred_element_type=jnp.float32)

    @pl.when(i == 0)
    def _set():
        csum_ref[...] = part

    @pl.when(i > 0)
    def _acc():
        csum_ref[...] = csum_ref[...] + part


def _pass_b_body(adj_ref, s2_ref, csum_ref, b2_ref, w3_ref, b3_ref, wl_ref,
                 bl_ref, out_ref, acc_scr, *, n_nodes):
    i = pl.program_id(0)
    h2 = jnp.maximum(
        jnp.dot(adj_ref[...], s2_ref[...],
                preferred_element_type=jnp.float32)
        + b2_ref[...], 0.0)
    contrib = jnp.dot(csum_ref[0], h2, preferred_element_type=jnp.float32)

    @pl.when(i == 0)
    def _set():
        acc_scr[...] = contrib

    @pl.when(i > 0)
    def _acc():
        acc_scr[...] = acc_scr[...] + contrib

    @pl.when(i == pl.num_programs(0) - 1)
    def _emit():
        m = acc_scr[...] * (1.0 / n_nodes)
        s3 = jnp.dot(m, w3_ref[...],
                     preferred_element_type=jnp.float32) + b3_ref[...]
        out_ref[...] = jnp.dot(s3, wl_ref[...],
                               preferred_element_type=jnp.float32) + bl_ref[...]


@jax.jit
def kernel(x, adj, W1, b1, W2, b2, W3, b3, Wl, bl):
    n, nfeat = x.shape
    nhid = W1.shape[1]
    nclasses = Wl.shape[1]
    bm = _BM
    grid = (n // bm,)

    b1r = b1.reshape(1, nhid)
    b2r = b2.reshape(1, nhid)
    b3r = b3.reshape(1, nhid)
    blr = bl.reshape(1, nclasses)

    full = lambda shape: pl.BlockSpec(shape, lambda i: (0,) * len(shape))

    s2, csum = pl.pallas_call(
        _pass_a_body,
        grid=grid,
        in_specs=[
            full((n, nfeat)),
            pl.BlockSpec((bm, n), lambda i: (i, 0)),
            full((nfeat, nhid)),
            full((1, nhid)),
            full((nhid, nhid)),
        ],
        out_specs=[
            pl.BlockSpec((bm, nhid), lambda i: (i, 0)),
            full((1, n)),
        ],
        out_shape=[
            jax.ShapeDtypeStruct((n, nhid), jnp.float32),
            jax.ShapeDtypeStruct((1, n), jnp.float32),
        ],
        scratch_shapes=[pltpu.VMEM((n, nhid), jnp.float32)],
        compiler_params=pltpu.CompilerParams(vmem_limit_bytes=_VMEM_LIMIT),
    )(x, adj, W1, b1r, W2)

    csum3 = csum.reshape(n // bm, 1, bm)

    out = pl.pallas_call(
        functools.partial(_pass_b_body, n_nodes=n),
        grid=grid,
        in_specs=[
            pl.BlockSpec((bm, n), lambda i: (i, 0)),
            full((n, nhid)),
            pl.BlockSpec((1, 1, bm), lambda i: (i, 0, 0)),
            full((1, nhid)),
            full((nhid, nhid)),
            full((1, nhid)),
            full((nhid, nclasses)),
            full((1, nclasses)),
        ],
        out_specs=full((1, nclasses)),
        out_shape=jax.ShapeDtypeStruct((1, nclasses), jnp.float32),
        scratch_shapes=[pltpu.VMEM((1, nhid), jnp.float32)],
        compiler_params=pltpu.CompilerParams(vmem_limit_bytes=_VMEM_LIMIT),
    )(adj, s2, csum3, b2r, W3, b3r, Wl, blr)

    return out


# speedup vs baseline: 1.1323x; 1.1146x over previous
import functools

import jax
import jax.numpy as jnp
from jax.experimental import pallas as pl
from jax.experimental.pallas import tpu as pltpu

_BM = 400
_VMEM_LIMIT = 64 * 1024 * 1024


def _pass_a_body(x_ref, adj_ref, w1_ref, b1_ref, w2_ref, s2_ref, s1_scr):
    i = pl.program_id(0)

    @pl.when(i == 0)
    def _init():
        s1_scr[...] = jnp.dot(x_ref[...], w1_ref[...],
                              preferred_element_type=jnp.float32)

    h1 = jnp.maximum(
        jnp.dot(adj_ref[...], s1_scr[...], preferred_element_type=jnp.float32)
        + b1_ref[...], 0.0)
    s2_ref[...] = jnp.dot(h1, w2_ref[...],
                          preferred_element_type=jnp.float32)


def _pass_b_body(adj_ref, s2_ref, b2_ref, w3_ref, b3_ref, wl_ref, bl_ref,
                 out_ref, h2_scr, csum_scr, *, bm, n_nodes):
    i = pl.program_id(0)
    adj_blk = adj_ref[...]
    h2_scr[pl.ds(i * bm, bm), :] = jnp.maximum(
        jnp.dot(adj_blk, s2_ref[...], preferred_element_type=jnp.float32)
        + b2_ref[...], 0.0)

    ones = jnp.ones((1, bm), dtype=jnp.float32)
    part = jnp.dot(ones, adj_blk, preferred_element_type=jnp.float32)

    @pl.when(i == 0)
    def _set():
        csum_scr[...] = part

    @pl.when(i > 0)
    def _acc():
        csum_scr[...] = csum_scr[...] + part

    @pl.when(i == pl.num_programs(0) - 1)
    def _emit():
        m = jnp.dot(csum_scr[...], h2_scr[...],
                    preferred_element_type=jnp.float32) * (1.0 / n_nodes)
        s3 = jnp.dot(m, w3_ref[...],
                     preferred_element_type=jnp.float32) + b3_ref[...]
        out_ref[...] = jnp.dot(s3, wl_ref[...],
                               preferred_element_type=jnp.float32) + bl_ref[...]


@jax.jit
def kernel(x, adj, W1, b1, W2, b2, W3, b3, Wl, bl):
    n, nfeat = x.shape
    nhid = W1.shape[1]
    nclasses = Wl.shape[1]
    bm = _BM
    grid = (n // bm,)

    b1r = b1.reshape(1, nhid)
    b2r = b2.reshape(1, nhid)
    b3r = b3.reshape(1, nhid)
    blr = bl.reshape(1, nclasses)

    full = lambda shape: pl.BlockSpec(shape, lambda i: (0,) * len(shape))

    s2 = pl.pallas_call(
        _pass_a_body,
        grid=grid,
        in_specs=[
            full((n, nfeat)),
            pl.BlockSpec((bm, n), lambda i: (i, 0)),
            full((nfeat, nhid)),
            full((1, nhid)),
            full((nhid, nhid)),
        ],
        out_specs=pl.BlockSpec((bm, nhid), lambda i: (i, 0)),
        out_shape=jax.ShapeDtypeStruct((n, nhid), jnp.float32),
        scratch_shapes=[pltpu.VMEM((n, nhid), jnp.float32)],
        compiler_params=pltpu.CompilerParams(vmem_limit_bytes=_VMEM_LIMIT),
    )(x, adj, W1, b1r, W2)

    out = pl.pallas_call(
        functools.partial(_pass_b_body, bm=bm, n_nodes=n),
        grid=grid,
        in_specs=[
            pl.BlockSpec((bm, n), lambda i: (i, 0)),
            full((n, nhid)),
            full((1, nhid)),
            full((nhid, nhid)),
            full((1, nhid)),
            full((nhid, nclasses)),
            full((1, nclasses)),
        ],
        out_specs=full((1, nclasses)),
        out_shape=jax.ShapeDtypeStruct((1, nclasses), jnp.float32),
        scratch_shapes=[
            pltpu.VMEM((n, nhid), jnp.float32),
            pltpu.VMEM((1, n), jnp.float32),
        ],
        compiler_params=pltpu.CompilerParams(vmem_limit_bytes=_VMEM_LIMIT),
    )(adj, s2, b2r, W3, b3r, Wl, blr)

    return out
